# baseline (speedup 1.0000x reference)
import functools
import math

import jax
import jax.numpy as jnp
from jax import lax
from jax.experimental import pallas as pl
from jax.experimental.pallas import tpu as pltpu
from jax.experimental.pallas import tpu_sc as plsc

_VOCAB = 100000
_D = 128
_BATCH = 4096
_SEQ = 50
_B = _BATCH * _SEQ
_SCALE = math.sqrt(_D)

_NC = 2
_NS = 16
_NW = _NC * _NS
_BPW = _BATCH // _NW
_NB = 4
_ROWS = _NB * _SEQ
_NCHUNK = _BPW // _NB


@functools.partial(
    pl.kernel,
    mesh=plsc.VectorSubcoreMesh(core_axis_name="c", subcore_axis_name="s"),
    out_type=jax.ShapeDtypeStruct((_BATCH, _SEQ, _D), jnp.float32),
    compiler_params=pltpu.CompilerParams(use_tc_tiling_on_sc=True),
    scratch_types=(
        [pltpu.VMEM((_ROWS,), jnp.int32) for _ in range(2)]
        + [pltpu.VMEM((_ROWS, _D), jnp.float32) for _ in range(2)]
        + [pltpu.SemaphoreType.DMA for _ in range(4)]
    ),
)
def _emb_lookup(tok_hbm, table_hbm, out_hbm, idx0, idx1, rows0, rows1,
                gsem0, gsem1, osem0, osem1):
    idx = (idx0, idx1)
    rows = (rows0, rows1)
    gsem = (gsem0, gsem1)
    osem = (osem0, osem1)

    wid = lax.axis_index("s") * _NC + lax.axis_index("c")
    base = wid * _BPW

    def start_gather(g, b):
        pltpu.sync_copy(
            tok_hbm.at[pl.ds((base + g * _NB) * _SEQ, _ROWS)], idx[b])
        pltpu.async_copy(table_hbm.at[idx[b]], rows[b], gsem[b])

    def wait_gather(b):
        pltpu.make_async_copy(
            table_hbm.at[pl.ds(0, _ROWS)], rows[b], gsem[b]).wait()

    def start_out(g, b):
        for u in range(_NB):
            pltpu.async_copy(
                rows[b].at[pl.ds(u * _SEQ, _SEQ)],
                out_hbm.at[base + g * _NB + u], osem[b])

    def wait_out(b):
        for _ in range(_NB):
            pltpu.make_async_copy(
                rows[b].at[pl.ds(0, _SEQ)], out_hbm.at[0], osem[b]).wait()

    def scale_buf(b):
        def scale_pair(i, c):
            for u in range(2):
                for j in range(_D // 16):
                    sl = pl.ds(j * 16, 16)
                    rows[b][i * 2 + u, sl] = rows[b][i * 2 + u, sl] * _SCALE
            return c
        lax.fori_loop(0, _ROWS // 2, scale_pair, 0)

    start_gather(0, 0)

    def pair_body(p, carry):
        g0 = p * 2
        for b in range(2):
            g = g0 + b
            nb = 1 - b
            @pl.when(g >= 1)
            def _():
                wait_out(nb)

            @pl.when(g + 1 < _NCHUNK)
            def _():
                start_gather(g + 1, nb)

            wait_gather(b)
            scale_buf(b)
            start_out(g, b)
        return carry

    lax.fori_loop(0, _NCHUNK // 2, pair_body, 0)
    wait_out(1)


def kernel(text_tokens, table):
    flat_tok = text_tokens.reshape(_B).astype(jnp.int32)
    return _emb_lookup(flat_tok, table)


# speedup vs baseline: 1.0029x; 1.0029x over previous
import functools
import math

import jax
import jax.numpy as jnp
from jax import lax
from jax.experimental import pallas as pl
from jax.experimental.pallas import tpu as pltpu
from jax.experimental.pallas import tpu_sc as plsc

_VOCAB = 100000
_D = 128
_BATCH = 4096
_SEQ = 50
_B = _BATCH * _SEQ
_SCALE = math.sqrt(_D)

_NC = 2
_NS = 16
_NW = _NC * _NS
_BPW = _BATCH // _NW
_NB = 4
_ROWS = _NB * _SEQ
_NCHUNK = _BPW // _NB


@functools.partial(
    pl.kernel,
    mesh=plsc.VectorSubcoreMesh(core_axis_name="c", subcore_axis_name="s"),
    out_type=jax.ShapeDtypeStruct((_BATCH, _SEQ, _D), jnp.float32),
    compiler_params=pltpu.CompilerParams(
        use_tc_tiling_on_sc=True, needs_layout_passes=True),
    scratch_types=(
        [pltpu.VMEM((_ROWS,), jnp.int32) for _ in range(2)]
        + [pltpu.VMEM((_ROWS, _D), jnp.float32) for _ in range(2)]
        + [pltpu.SemaphoreType.DMA for _ in range(4)]
    ),
)
def _emb_lookup(tok_hbm, table_hbm, out_hbm, idx0, idx1, rows0, rows1,
                gsem0, gsem1, osem0, osem1):
    idx = (idx0, idx1)
    rows = (rows0, rows1)
    gsem = (gsem0, gsem1)
    osem = (osem0, osem1)

    wid = lax.axis_index("s") * _NC + lax.axis_index("c")
    base = wid * _BPW

    def start_gather(g, b):
        pltpu.sync_copy(
            tok_hbm.at[pl.ds((base + g * _NB) * _SEQ, _ROWS)], idx[b])
        pltpu.async_copy(table_hbm.at[idx[b]], rows[b], gsem[b])

    def wait_gather(b):
        pltpu.make_async_copy(
            table_hbm.at[pl.ds(0, _ROWS)], rows[b], gsem[b]).wait()

    def start_out(g, b):
        for u in range(_NB):
            pltpu.async_copy(
                rows[b].at[pl.ds(u * _SEQ, _SEQ)],
                out_hbm.at[base + g * _NB + u], osem[b])

    def wait_out(b):
        for _ in range(_NB):
            pltpu.make_async_copy(
                rows[b].at[pl.ds(0, _SEQ)], out_hbm.at[0], osem[b]).wait()

    def scale_buf(b):
        def scale_pair(i, c):
            for u in range(2):
                for j in range(_D // 16):
                    sl = pl.ds(j * 16, 16)
                    rows[b][i * 2 + u, sl] = rows[b][i * 2 + u, sl] * _SCALE
            return c
        lax.fori_loop(0, _ROWS // 2, scale_pair, 0)

    start_gather(0, 0)

    def pair_body(p, carry):
        g0 = p * 2
        for b in range(2):
            g = g0 + b
            nb = 1 - b
            @pl.when(g >= 1)
            def _():
                wait_out(nb)

            @pl.when(g + 1 < _NCHUNK)
            def _():
                start_gather(g + 1, nb)

            wait_gather(b)
            scale_buf(b)
            start_out(g, b)
        return carry

    lax.fori_loop(0, _NCHUNK // 2, pair_body, 0)
    wait_out(1)


def kernel(text_tokens, table):
    flat_tok = text_tokens.reshape(_B).astype(jnp.int32)
    return _emb_lookup(flat_tok, table)
